# Initial kernel scaffold; baseline (speedup 1.0000x reference)
import jax
import jax.numpy as jnp
from jax.experimental import pallas as pl


def kernel(query_entities, query_relations, obj_entities, ent_table, rel_table):
    raise NotImplementedError("write your pallas kernel here")


# speedup vs baseline: 2.4494x; 2.4494x over previous
import functools

import jax
import jax.numpy as jnp
from jax import lax
from jax.experimental import pallas as pl
from jax.experimental.pallas import tpu as pltpu
from jax.experimental.pallas import tpu_sc as plsc

_B = 16384
_D = 128
_CHUNK = 128


def _build():
    info = plsc.get_sparse_core_info()
    nc, ns = info.num_cores, info.num_subcores
    nw = nc * ns
    b_per_w = _B // nw
    n_chunks = b_per_w // _CHUNK
    mesh = plsc.VectorSubcoreMesh(core_axis_name="c", subcore_axis_name="s")
    out_t = jax.ShapeDtypeStruct((_B, _D), jnp.float32)

    @functools.partial(
        pl.kernel,
        out_type=(out_t, out_t, out_t),
        mesh=mesh,
        scratch_types=[
            pltpu.VMEM((3 * n_chunks, _CHUNK), jnp.int32),
            pltpu.VMEM((_CHUNK, _D), jnp.float32),
            pltpu.VMEM((_CHUNK, _D), jnp.float32),
            pltpu.SemaphoreType.DMA,
            pltpu.SemaphoreType.DMA,
        ],
    )
    def k(qe_hbm, qr_hbm, oe_hbm, ent_hbm, rel_hbm,
          out_qe, out_qr, out_oe,
          idx_v, rows0, rows1, sem0, sem1):
        wid = lax.axis_index("s") * nc + lax.axis_index("c")
        idx_row0 = wid * n_chunks
        pltpu.sync_copy(qe_hbm.at[pl.ds(idx_row0, n_chunks)],
                        idx_v.at[pl.ds(0, n_chunks)])
        pltpu.sync_copy(qr_hbm.at[pl.ds(idx_row0, n_chunks)],
                        idx_v.at[pl.ds(n_chunks, n_chunks)])
        pltpu.sync_copy(oe_hbm.at[pl.ds(idx_row0, n_chunks)],
                        idx_v.at[pl.ds(2 * n_chunks, n_chunks)])

        tables = (ent_hbm, rel_hbm, ent_hbm)
        outs = (out_qe, out_qr, out_oe)
        bufs = (rows0, rows1)
        sems = (sem0, sem1)
        base = wid * b_per_w

        chunks = []
        for g in range(3):
            for j in range(n_chunks):
                chunks.append((g * n_chunks + j, tables[g], outs[g],
                               base + j * _CHUNK))

        def gather(t, b):
            ir, tbl, _, _ = chunks[t]
            return pltpu.make_async_copy(tbl.at[idx_v.at[ir]], bufs[b],
                                         sems[b])

        gather(0, 0).start()
        for t in range(len(chunks)):
            b = t % 2
            if t + 1 < len(chunks):
                gather(t + 1, (t + 1) % 2).start()
            gather(t, b).wait()
            _, _, out_ref, off = chunks[t]
            pltpu.sync_copy(bufs[b], out_ref.at[pl.ds(off, _CHUNK)])

    return k


_kernel_fn = _build()


def kernel(query_entities, query_relations, obj_entities, ent_table, rel_table):
    n_rows = _B // _CHUNK
    qe = query_entities.reshape(n_rows, _CHUNK)
    qr = query_relations.reshape(n_rows, _CHUNK)
    oe = obj_entities.reshape(n_rows, _CHUNK)
    return _kernel_fn(qe, qr, oe, ent_table, rel_table)


# speedup vs baseline: 2.5554x; 1.0433x over previous
import functools

import jax
import jax.numpy as jnp
from jax import lax
from jax.experimental import pallas as pl
from jax.experimental.pallas import tpu as pltpu
from jax.experimental.pallas import tpu_sc as plsc

_B = 16384
_D = 128
_CHUNK = 128


_NBUF = 4


def _build():
    info = plsc.get_sparse_core_info()
    nc, ns = info.num_cores, info.num_subcores
    nw = nc * ns
    b_per_w = _B // nw
    n_chunks = b_per_w // _CHUNK
    n_t = 3 * n_chunks
    mesh = plsc.VectorSubcoreMesh(core_axis_name="c", subcore_axis_name="s")
    out_t = jax.ShapeDtypeStruct((_B, _D), jnp.float32)

    @functools.partial(
        pl.kernel,
        out_type=(out_t, out_t, out_t),
        mesh=mesh,
        scratch_types=[
            pltpu.VMEM((n_t, _CHUNK), jnp.int32),
        ] + [pltpu.VMEM((_CHUNK, _D), jnp.float32)] * _NBUF
          + [pltpu.SemaphoreType.DMA] * (2 * _NBUF),
    )
    def k(idx_hbm, ent_hbm, rel_hbm, out_qe, out_qr, out_oe, idx_v, *rest):
        bufs = rest[:_NBUF]
        gsems = rest[_NBUF:2 * _NBUF]
        wsems = rest[2 * _NBUF:]
        wid = lax.axis_index("s") * nc + lax.axis_index("c")
        pltpu.sync_copy(idx_hbm.at[wid], idx_v)

        tables = (ent_hbm, rel_hbm, ent_hbm)
        outs = (out_qe, out_qr, out_oe)
        base = wid * b_per_w

        def gather(t):
            g, j = divmod(t, n_chunks)
            return pltpu.make_async_copy(
                tables[g].at[idx_v.at[t]], bufs[t % _NBUF],
                gsems[t % _NBUF])

        def writeout(t):
            g, j = divmod(t, n_chunks)
            return pltpu.make_async_copy(
                bufs[t % _NBUF],
                outs[g].at[pl.ds(base + j * _CHUNK, _CHUNK)],
                wsems[t % _NBUF])

        for t in range(_NBUF):
            gather(t).start()
        for t in range(n_t):
            gather(t).wait()
            writeout(t).start()
            if t + _NBUF < n_t:
                writeout(t).wait()
                gather(t + _NBUF).start()
        for t in range(n_t - _NBUF, n_t):
            writeout(t).wait()

    return k


_kernel_fn = _build()


def kernel(query_entities, query_relations, obj_entities, ent_table, rel_table):
    nw = 32
    per_w = _B // nw // _CHUNK
    idx = jnp.stack([
        query_entities.reshape(nw, per_w, _CHUNK),
        query_relations.reshape(nw, per_w, _CHUNK),
        obj_entities.reshape(nw, per_w, _CHUNK),
    ], axis=1).reshape(nw, 3 * per_w, _CHUNK)
    return _kernel_fn(idx, ent_table, rel_table)


# speedup vs baseline: 2.6201x; 1.0253x over previous
import functools

import jax
import jax.numpy as jnp
from jax import lax
from jax.experimental import pallas as pl
from jax.experimental.pallas import tpu as pltpu
from jax.experimental.pallas import tpu_sc as plsc

_B = 16384
_D = 128
_CHUNK = 128


_NBUF = 6
_LAG = 2


def _build():
    info = plsc.get_sparse_core_info()
    nc, ns = info.num_cores, info.num_subcores
    nw = nc * ns
    b_per_w = _B // nw
    n_chunks = b_per_w // _CHUNK
    n_t = 3 * n_chunks
    mesh = plsc.VectorSubcoreMesh(core_axis_name="c", subcore_axis_name="s")
    out_t = jax.ShapeDtypeStruct((_B, _D), jnp.float32)

    @functools.partial(
        pl.kernel,
        out_type=(out_t, out_t, out_t),
        mesh=mesh,
        scratch_types=[
            pltpu.VMEM((n_t, _CHUNK), jnp.int32),
        ] + [pltpu.VMEM((_CHUNK, _D), jnp.float32)] * _NBUF
          + [pltpu.SemaphoreType.DMA] * (2 * _NBUF),
    )
    def k(idx_hbm, ent_hbm, rel_hbm, out_qe, out_qr, out_oe, idx_v, *rest):
        bufs = rest[:_NBUF]
        gsems = rest[_NBUF:2 * _NBUF]
        wsems = rest[2 * _NBUF:]
        wid = lax.axis_index("s") * nc + lax.axis_index("c")
        pltpu.sync_copy(idx_hbm.at[wid], idx_v)

        tables = (ent_hbm, rel_hbm, ent_hbm)
        outs = (out_qe, out_qr, out_oe)
        base = wid * b_per_w

        def gather(t):
            g, j = divmod(t, n_chunks)
            return pltpu.make_async_copy(
                tables[g].at[idx_v.at[t]], bufs[t % _NBUF],
                gsems[t % _NBUF])

        def writeout(t):
            g, j = divmod(t, n_chunks)
            return pltpu.make_async_copy(
                bufs[t % _NBUF],
                outs[g].at[pl.ds(base + j * _CHUNK, _CHUNK)],
                wsems[t % _NBUF])

        for t in range(_NBUF):
            gather(t).start()
        for t in range(n_t):
            gather(t).wait()
            writeout(t).start()
            s = t - _LAG
            if s >= 0 and s + _NBUF < n_t:
                writeout(s).wait()
                gather(s + _NBUF).start()
        for t in range(n_t - _NBUF, n_t):
            writeout(t).wait()

    return k


_kernel_fn = _build()


def kernel(query_entities, query_relations, obj_entities, ent_table, rel_table):
    nw = 32
    per_w = _B // nw // _CHUNK
    idx = jnp.stack([
        query_entities.reshape(nw, per_w, _CHUNK),
        query_relations.reshape(nw, per_w, _CHUNK),
        obj_entities.reshape(nw, per_w, _CHUNK),
    ], axis=1).reshape(nw, 3 * per_w, _CHUNK)
    return _kernel_fn(idx, ent_table, rel_table)
